# baseline (speedup 1.0000x reference)
import functools

import jax
import jax.numpy as jnp
from jax import lax
from jax.experimental import pallas as pl
from jax.experimental.pallas import tpu as pltpu
from jax.experimental.pallas import tpu_sc as plsc

VOCAB = 100000
EMBED_DIM = 64
BATCH = 16384
HIST = 50

NUM_CORES = 2
NUM_SUBCORES = 16
NUM_WORKERS = NUM_CORES * NUM_SUBCORES
BAGS_PER_WORKER = BATCH // NUM_WORKERS
CHUNK = 8
NUM_CHUNKS = BAGS_PER_WORKER // CHUNK
ROWS = CHUNK * HIST
NLANE = 16
DCHUNKS = EMBED_DIM // NLANE

_mesh = plsc.VectorSubcoreMesh(
    core_axis_name="c", subcore_axis_name="s",
    num_cores=NUM_CORES, num_subcores=NUM_SUBCORES)


@functools.partial(
    pl.kernel,
    mesh=_mesh,
    out_type=jax.ShapeDtypeStruct((BATCH * EMBED_DIM,), jnp.float32),
    scratch_types=[
        pltpu.VMEM((4, ROWS), jnp.int32),
        pltpu.VMEM((ROWS, EMBED_DIM), jnp.float32),
        pltpu.VMEM((ROWS, EMBED_DIM), jnp.float32),
        pltpu.VMEM((ROWS, EMBED_DIM), jnp.float32),
        pltpu.VMEM((ROWS, EMBED_DIM), jnp.float32),
        pltpu.VMEM((CHUNK * EMBED_DIM,), jnp.float32),
        pltpu.VMEM((CHUNK * EMBED_DIM,), jnp.float32),
        pltpu.SemaphoreType.DMA,
        pltpu.SemaphoreType.DMA,
        pltpu.SemaphoreType.DMA,
        pltpu.SemaphoreType.DMA,
        pltpu.SemaphoreType.DMA,
        pltpu.SemaphoreType.DMA,
        pltpu.SemaphoreType.DMA,
        pltpu.SemaphoreType.DMA,
        pltpu.SemaphoreType.DMA,
        pltpu.SemaphoreType.DMA,
    ],
    compiler_params=pltpu.CompilerParams(use_tc_tiling_on_sc=False),
)
def _embed_bag_mean(idx_hbm, w_hbm, out_hbm, idx_v, rows0_v, rows1_v,
                    rows2_v, rows3_v, out0_v, out1_v, gsem0, gsem1, gsem2,
                    gsem3, osem0, osem1, isem0, isem1, isem2, isem3):
    wid = lax.axis_index("s") * NUM_CORES + lax.axis_index("c")
    base_bag = wid * BAGS_PER_WORKER
    rows_bufs = (rows0_v, rows1_v, rows2_v, rows3_v)
    out_bufs = (out0_v, out1_v)
    gsems = (gsem0, gsem1, gsem2, gsem3)
    osems = (osem0, osem1)
    isems = (isem0, isem1, isem2, isem3)

    def idx_slice(ci):
        return idx_hbm.at[pl.ds((base_bag + ci * CHUNK) * HIST, ROWS)]

    for p in range(3):
        pltpu.sync_copy(idx_slice(p), idx_v.at[p])
        pltpu.async_copy(w_hbm.at[idx_v.at[p]], rows_bufs[p], gsems[p])
    for p in range(3, 4):
        pltpu.async_copy(idx_slice(p), idx_v.at[p], isems[p])

    def process(rows_v, out_v):
        def bag_body(bi, carry2):
            acc = [rows_v[bi * HIST, pl.ds(c * NLANE, NLANE)]
                   for c in range(DCHUNKS)]
            for ri in range(1, HIST):
                for c in range(DCHUNKS):
                    acc[c] = acc[c] + rows_v[bi * HIST + ri,
                                             pl.ds(c * NLANE, NLANE)]
            for c in range(DCHUNKS):
                out_v[pl.ds(bi * EMBED_DIM + c * NLANE, NLANE)] = jnp.maximum(
                    acc[c] * (1.0 / HIST), 0.0)
            return carry2

        lax.fori_loop(0, CHUNK, bag_body, 0)

    def outer(ci4, carry):
        for b in range(4):
            ci = ci4 * 4 + b
            nb = (b + 3) % 4
            ob = b % 2

            @pl.when(ci + 3 < NUM_CHUNKS)
            def _():
                pltpu.make_async_copy(idx_slice(ci + 3), idx_v.at[nb],
                                      isems[nb]).wait()
                pltpu.async_copy(w_hbm.at[idx_v.at[nb]], rows_bufs[nb],
                                 gsems[nb])

            pltpu.make_async_copy(w_hbm.at[idx_v.at[b]],
                                  rows_bufs[b], gsems[b]).wait()

            @pl.when(ci + 4 < NUM_CHUNKS)
            def _():
                pltpu.async_copy(idx_slice(ci + 4), idx_v.at[b], isems[b])

            @pl.when(ci >= 2)
            def _():
                pltpu.make_async_copy(
                    out_bufs[ob],
                    out_hbm.at[pl.ds(base_bag * EMBED_DIM, CHUNK * EMBED_DIM)],
                    osems[ob]).wait()

            process(rows_bufs[b], out_bufs[ob])
            pltpu.async_copy(
                out_bufs[ob],
                out_hbm.at[pl.ds((base_bag + ci * CHUNK) * EMBED_DIM,
                                 CHUNK * EMBED_DIM)],
                osems[ob])
        return carry

    lax.fori_loop(0, NUM_CHUNKS // 4, outer, 0)

    for b in range(2):
        pltpu.make_async_copy(
            out_bufs[b],
            out_hbm.at[pl.ds(base_bag * EMBED_DIM, CHUNK * EMBED_DIM)],
            osems[b]).wait()


def kernel(indices, W):
    idx2 = (indices.astype(jnp.int32) * 2).reshape(-1)
    w_pad = jnp.pad(W, ((0, 0), (0, EMBED_DIM))).reshape(2 * VOCAB, EMBED_DIM)
    out_flat = _embed_bag_mean(idx2, w_pad)
    return out_flat.reshape(BATCH, EMBED_DIM)


# speedup vs baseline: 1.0076x; 1.0076x over previous
import functools

import jax
import jax.numpy as jnp
from jax import lax
from jax.experimental import pallas as pl
from jax.experimental.pallas import tpu as pltpu
from jax.experimental.pallas import tpu_sc as plsc

VOCAB = 100000
EMBED_DIM = 64
BATCH = 16384
HIST = 50

NUM_CORES = 2
NUM_SUBCORES = 16
NUM_WORKERS = NUM_CORES * NUM_SUBCORES
BAGS_PER_WORKER = BATCH // NUM_WORKERS
CHUNK = 8
NUM_CHUNKS = BAGS_PER_WORKER // CHUNK
ROWS = CHUNK * HIST
NLANE = 16
DCHUNKS = EMBED_DIM // NLANE

_mesh = plsc.VectorSubcoreMesh(
    core_axis_name="c", subcore_axis_name="s",
    num_cores=NUM_CORES, num_subcores=NUM_SUBCORES)


@functools.partial(
    pl.kernel,
    mesh=_mesh,
    out_type=jax.ShapeDtypeStruct((BATCH * EMBED_DIM,), jnp.float32),
    scratch_types=[
        pltpu.VMEM((4, ROWS), jnp.int32),
        pltpu.VMEM((ROWS, EMBED_DIM), jnp.float32),
        pltpu.VMEM((ROWS, EMBED_DIM), jnp.float32),
        pltpu.VMEM((ROWS, EMBED_DIM), jnp.float32),
        pltpu.VMEM((ROWS, EMBED_DIM), jnp.float32),
        pltpu.VMEM((CHUNK * EMBED_DIM,), jnp.float32),
        pltpu.VMEM((CHUNK * EMBED_DIM,), jnp.float32),
        pltpu.SemaphoreType.DMA,
        pltpu.SemaphoreType.DMA,
        pltpu.SemaphoreType.DMA,
        pltpu.SemaphoreType.DMA,
        pltpu.SemaphoreType.DMA,
        pltpu.SemaphoreType.DMA,
        pltpu.SemaphoreType.DMA,
        pltpu.SemaphoreType.DMA,
        pltpu.SemaphoreType.DMA,
        pltpu.SemaphoreType.DMA,
    ],
    compiler_params=pltpu.CompilerParams(use_tc_tiling_on_sc=False),
)
def _embed_bag_mean(idx_hbm, w_hbm, out_hbm, idx_v, rows0_v, rows1_v,
                    rows2_v, rows3_v, out0_v, out1_v, gsem0, gsem1, gsem2,
                    gsem3, osem0, osem1, isem0, isem1, isem2, isem3):
    wid = lax.axis_index("s") * NUM_CORES + lax.axis_index("c")
    base_bag = wid * BAGS_PER_WORKER
    rows_bufs = (rows0_v, rows1_v, rows2_v, rows3_v)
    out_bufs = (out0_v, out1_v)
    gsems = (gsem0, gsem1, gsem2, gsem3)
    osems = (osem0, osem1)
    isems = (isem0, isem1, isem2, isem3)

    def idx_slice(ci):
        return idx_hbm.at[pl.ds((base_bag + ci * CHUNK) * HIST, ROWS)]

    for p in range(2):
        pltpu.sync_copy(idx_slice(p), idx_v.at[p])
        pltpu.async_copy(w_hbm.at[idx_v.at[p]], rows_bufs[p], gsems[p])
    for p in range(2, 4):
        pltpu.async_copy(idx_slice(p), idx_v.at[p], isems[p])

    def process(rows_v, out_v):
        def bag_body(bi, carry2):
            acc = [rows_v[bi * HIST, pl.ds(c * NLANE, NLANE)]
                   for c in range(DCHUNKS)]
            for ri in range(1, HIST):
                for c in range(DCHUNKS):
                    acc[c] = acc[c] + rows_v[bi * HIST + ri,
                                             pl.ds(c * NLANE, NLANE)]
            for c in range(DCHUNKS):
                out_v[pl.ds(bi * EMBED_DIM + c * NLANE, NLANE)] = jnp.maximum(
                    acc[c] * (1.0 / HIST), 0.0)
            return carry2

        lax.fori_loop(0, CHUNK, bag_body, 0)

    def outer(ci4, carry):
        for b in range(4):
            ci = ci4 * 4 + b
            nb = (b + 2) % 4
            ob = b % 2

            @pl.when(ci + 2 < NUM_CHUNKS)
            def _():
                pltpu.make_async_copy(idx_slice(ci + 2), idx_v.at[nb],
                                      isems[nb]).wait()
                pltpu.async_copy(w_hbm.at[idx_v.at[nb]], rows_bufs[nb],
                                 gsems[nb])

            pltpu.make_async_copy(w_hbm.at[idx_v.at[b]],
                                  rows_bufs[b], gsems[b]).wait()

            @pl.when(ci + 4 < NUM_CHUNKS)
            def _():
                pltpu.async_copy(idx_slice(ci + 4), idx_v.at[b], isems[b])

            @pl.when(ci >= 2)
            def _():
                pltpu.make_async_copy(
                    out_bufs[ob],
                    out_hbm.at[pl.ds(base_bag * EMBED_DIM, CHUNK * EMBED_DIM)],
                    osems[ob]).wait()

            process(rows_bufs[b], out_bufs[ob])
            pltpu.async_copy(
                out_bufs[ob],
                out_hbm.at[pl.ds((base_bag + ci * CHUNK) * EMBED_DIM,
                                 CHUNK * EMBED_DIM)],
                osems[ob])
        return carry

    lax.fori_loop(0, NUM_CHUNKS // 4, outer, 0)

    for b in range(2):
        pltpu.make_async_copy(
            out_bufs[b],
            out_hbm.at[pl.ds(base_bag * EMBED_DIM, CHUNK * EMBED_DIM)],
            osems[b]).wait()


def kernel(indices, W):
    idx2 = (indices.astype(jnp.int32) * 2).reshape(-1)
    w_pad = jnp.pad(W, ((0, 0), (0, EMBED_DIM))).reshape(2 * VOCAB, EMBED_DIM)
    out_flat = _embed_bag_mean(idx2, w_pad)
    return out_flat.reshape(BATCH, EMBED_DIM)
